# Initial kernel scaffold; baseline (speedup 1.0000x reference)
import functools

import jax
import jax.numpy as jnp
from jax import lax
from jax.experimental import pallas as pl
from jax.experimental.pallas import tpu as pltpu
from jax.experimental.pallas import tpu_sc as plsc

B, N, K, H, V = 8, 128, 8, 32, 512
NW = 32
ROWS_PER_W = (B * N) // NW


def _sc_edge_body(tab_hbm, ed_hbm, out_hbm, tab_v, idx_v, slab_v, d32_v):
    wid = lax.axis_index("s") * 2 + lax.axis_index("c")
    pltpu.sync_copy(tab_hbm, tab_v)
    lane = lax.iota(jnp.int32, 16)
    lane8 = lane * 8

    def row_body(r, _):
        row = wid * ROWS_PER_W + r
        b = row // N
        i = row % N
        pltpu.sync_copy(ed_hbm.at[row], idx_v)

        def g_body(g, _):
            for k in range(K):
                dk = plsc.load_gather(idx_v, [lane8 + (g * 128 + k)])
                d32_v[pl.ds(k * 16, 16)] = dk * H

            def h_body(h, _):
                accs = []
                for k in range(K):
                    d32 = d32_v[pl.ds(k * 16, 16)]
                    accs.append(plsc.load_gather(tab_v, [d32 + h]))
                s = ((accs[0] + accs[1]) + (accs[2] + accs[3])) + (
                    (accs[4] + accs[5]) + (accs[6] + accs[7]))
                plsc.store_scatter(
                    slab_v,
                    [jnp.full((16,), h, jnp.int32), lane + g * 16],
                    s * 0.125,
                )
                return 0

            lax.fori_loop(0, H, h_body, 0)
            return 0

        lax.fori_loop(0, N // 16, g_body, 0)
        pltpu.sync_copy(slab_v, out_hbm.at[b, :, i, :])
        return 0

    lax.fori_loop(0, ROWS_PER_W, row_body, 0)


@jax.jit
def _sc_edge(tab_flat, ed_rows):
    mesh = plsc.VectorSubcoreMesh(core_axis_name="c", subcore_axis_name="s")
    return pl.kernel(
        _sc_edge_body,
        out_type=jax.ShapeDtypeStruct((B, H, N, N), jnp.float32),
        mesh=mesh,
        scratch_types=[
            pltpu.VMEM((V * H,), jnp.float32),
            pltpu.VMEM((N * K,), jnp.int32),
            pltpu.VMEM((H, N), jnp.float32),
            pltpu.VMEM((K * 16,), jnp.int32),
        ],
    )(tab_flat, ed_rows)


def _tc_assemble_body(ab_ref, e_ref, o_ref):
    ab = ab_ref[0]
    e = e_ref[0]
    base = 2.0 * jnp.broadcast_to(ab[None, :, :], (H, N + 1, N + 1))
    row0 = jnp.zeros((H, 1, N), jnp.float32)
    col0 = jnp.zeros((H, N + 1, 1), jnp.float32)
    padded = jnp.concatenate(
        [col0, jnp.concatenate([row0, e], axis=1)], axis=2)
    o_ref[0] = base + padded


@jax.jit
def _tc_assemble(attn_bias, edge):
    return pl.pallas_call(
        _tc_assemble_body,
        out_shape=jax.ShapeDtypeStruct((B, H, N + 1, N + 1), jnp.float32),
        grid=(B,),
        in_specs=[
            pl.BlockSpec((1, N + 1, N + 1), lambda b: (b, 0, 0)),
            pl.BlockSpec((1, H, N, N), lambda b: (b, 0, 0, 0)),
        ],
        out_specs=pl.BlockSpec((1, H, N + 1, N + 1), lambda b: (b, 0, 0, 0)),
    )(attn_bias, edge)


def kernel(attn_bias, edge_data, edge_encoder_weight):
    ed = edge_data.astype(jnp.int32).reshape(B * N, N * K)
    tab = edge_encoder_weight.astype(jnp.float32).reshape(V * H)
    edge = _sc_edge(tab, ed)
    return _tc_assemble(attn_bias, edge)


# speedup vs baseline: 4.8789x; 4.8789x over previous
import functools

import jax
import jax.numpy as jnp
from jax import lax
from jax.experimental import pallas as pl
from jax.experimental.pallas import tpu as pltpu
from jax.experimental.pallas import tpu_sc as plsc

B, N, K, H, V = 8, 128, 8, 32, 512
NW = 32
ROWS_PER_W = (B * N) // NW


def _sc_edge_body(tab_hbm, ed_hbm, out_hbm, tab_v, idx_v, slab_v, d32_v):
    wid = lax.axis_index("s") * 2 + lax.axis_index("c")
    pltpu.sync_copy(tab_hbm, tab_v)
    lane = lax.iota(jnp.int32, 16)
    lane8 = lane * 8

    def row_body(r, _):
        row = wid * ROWS_PER_W + r
        b = row // N
        i = row % N
        pltpu.sync_copy(ed_hbm.at[row], idx_v)

        def g_body(g, _):
            for k in range(K):
                dk = plsc.load_gather(idx_v, [lane8 + (g * 128 + k)])
                d32_v[pl.ds(k * 16, 16)] = dk * H

            def h_body(h, _):
                accs = []
                for k in range(K):
                    d32 = d32_v[pl.ds(k * 16, 16)]
                    accs.append(plsc.load_gather(tab_v, [d32 + h]))
                s = ((accs[0] + accs[1]) + (accs[2] + accs[3])) + (
                    (accs[4] + accs[5]) + (accs[6] + accs[7]))
                plsc.store_scatter(
                    slab_v,
                    [jnp.full((16,), h, jnp.int32), lane + g * 16],
                    s * 0.125,
                )
                return 0

            lax.fori_loop(0, H, h_body, 0)
            return 0

        lax.fori_loop(0, N // 16, g_body, 0)
        pltpu.sync_copy(slab_v, out_hbm.at[b, :, i, :])
        return 0

    lax.fori_loop(0, ROWS_PER_W, row_body, 0)


@jax.jit
def _sc_edge(tab_flat, ed_rows):
    mesh = plsc.VectorSubcoreMesh(core_axis_name="c", subcore_axis_name="s")
    return pl.kernel(
        _sc_edge_body,
        out_type=jax.ShapeDtypeStruct((B, H, N, N), jnp.float32),
        mesh=mesh,
        compiler_params=pltpu.CompilerParams(needs_layout_passes=False),
        scratch_types=[
            pltpu.VMEM((V * H,), jnp.float32),
            pltpu.VMEM((N * K,), jnp.int32),
            pltpu.VMEM((H, N), jnp.float32),
            pltpu.VMEM((K * 16,), jnp.int32),
        ],
    )(tab_flat, ed_rows)


def _tc_assemble_body(ab_ref, e_ref, o_ref):
    ab = ab_ref[0]
    e = e_ref[0]
    base = 2.0 * jnp.broadcast_to(ab[None, :, :], (H, N + 1, N + 1))
    row0 = jnp.zeros((H, 1, N), jnp.float32)
    col0 = jnp.zeros((H, N + 1, 1), jnp.float32)
    padded = jnp.concatenate(
        [col0, jnp.concatenate([row0, e], axis=1)], axis=2)
    o_ref[0] = base + padded


@jax.jit
def _tc_assemble(attn_bias, edge):
    return pl.pallas_call(
        _tc_assemble_body,
        out_shape=jax.ShapeDtypeStruct((B, H, N + 1, N + 1), jnp.float32),
        grid=(B,),
        in_specs=[
            pl.BlockSpec((1, N + 1, N + 1), lambda b: (b, 0, 0)),
            pl.BlockSpec((1, H, N, N), lambda b: (b, 0, 0, 0)),
        ],
        out_specs=pl.BlockSpec((1, H, N + 1, N + 1), lambda b: (b, 0, 0, 0)),
    )(attn_bias, edge)


def kernel(attn_bias, edge_data, edge_encoder_weight):
    ed = edge_data.astype(jnp.int32).reshape(B * N, N * K)
    tab = edge_encoder_weight.astype(jnp.float32).reshape(V * H)
    edge = _sc_edge(tab, ed)
    return _tc_assemble(attn_bias, edge)


# speedup vs baseline: 5.2463x; 1.0753x over previous
import functools

import jax
import jax.numpy as jnp
from jax import lax
from jax.experimental import pallas as pl
from jax.experimental.pallas import tpu as pltpu
from jax.experimental.pallas import tpu_sc as plsc

B, N, K, H, V = 8, 128, 8, 32, 512
NW = 32
ROWS_PER_W = (B * N) // NW


def _sc_edge_body(tab_hbm, ed_hbm, out_hbm, tab_v, idx_v, slab_v):
    wid = lax.axis_index("s") * 2 + lax.axis_index("c")
    pltpu.sync_copy(tab_hbm, tab_v)
    lane = lax.iota(jnp.int32, 16)
    lane8 = lane * 8

    def row_body(r, _):
        row = wid * ROWS_PER_W + r
        b = row // N
        i = row % N
        pltpu.sync_copy(ed_hbm.at[row], idx_v)

        def g_body(g, _):
            d32 = [
                plsc.load_gather(idx_v, [lane8 + (g * 128 + k)]) * H
                for k in range(K)
            ]

            def h_body(hi, _):
                for hh in range(4):
                    h = hi * 4 + hh
                    a = [plsc.load_gather(tab_v, [d32[k] + h])
                         for k in range(K)]
                    s = ((a[0] + a[1]) + (a[2] + a[3])) + (
                        (a[4] + a[5]) + (a[6] + a[7]))
                    plsc.store_scatter(
                        slab_v,
                        [jnp.full((16,), h, jnp.int32), lane + g * 16],
                        s * 0.125,
                    )
                return 0

            lax.fori_loop(0, H // 4, h_body, 0)
            return 0

        lax.fori_loop(0, N // 16, g_body, 0)
        pltpu.sync_copy(slab_v, out_hbm.at[b, :, i, :])
        return 0

    lax.fori_loop(0, ROWS_PER_W, row_body, 0)


@jax.jit
def _sc_edge(tab_flat, ed_rows):
    mesh = plsc.VectorSubcoreMesh(core_axis_name="c", subcore_axis_name="s")
    return pl.kernel(
        _sc_edge_body,
        out_type=jax.ShapeDtypeStruct((B, H, N, N), jnp.float32),
        mesh=mesh,
        compiler_params=pltpu.CompilerParams(needs_layout_passes=False),
        scratch_types=[
            pltpu.VMEM((V * H,), jnp.float32),
            pltpu.VMEM((N * K,), jnp.int32),
            pltpu.VMEM((H, N), jnp.float32),
        ],
    )(tab_flat, ed_rows)


def _tc_assemble_body(ab_ref, e_ref, o_ref):
    ab = ab_ref[0]
    e = e_ref[0]
    base = 2.0 * jnp.broadcast_to(ab[None, :, :], (H, N + 1, N + 1))
    row0 = jnp.zeros((H, 1, N), jnp.float32)
    col0 = jnp.zeros((H, N + 1, 1), jnp.float32)
    padded = jnp.concatenate(
        [col0, jnp.concatenate([row0, e], axis=1)], axis=2)
    o_ref[0] = base + padded


@jax.jit
def _tc_assemble(attn_bias, edge):
    return pl.pallas_call(
        _tc_assemble_body,
        out_shape=jax.ShapeDtypeStruct((B, H, N + 1, N + 1), jnp.float32),
        grid=(B,),
        in_specs=[
            pl.BlockSpec((1, N + 1, N + 1), lambda b: (b, 0, 0)),
            pl.BlockSpec((1, H, N, N), lambda b: (b, 0, 0, 0)),
        ],
        out_specs=pl.BlockSpec((1, H, N + 1, N + 1), lambda b: (b, 0, 0, 0)),
    )(attn_bias, edge)


def kernel(attn_bias, edge_data, edge_encoder_weight):
    ed = edge_data.astype(jnp.int32).reshape(B * N, N * K)
    tab = edge_encoder_weight.astype(jnp.float32).reshape(V * H)
    edge = _sc_edge(tab, ed)
    return _tc_assemble(attn_bias, edge)


# speedup vs baseline: 6.4835x; 1.2358x over previous
import functools

import jax
import jax.numpy as jnp
from jax import lax
from jax.experimental import pallas as pl
from jax.experimental.pallas import tpu as pltpu
from jax.experimental.pallas import tpu_sc as plsc

B, N, K, H, V = 8, 128, 8, 32, 512
NW = 32
ROWS_PER_W = (B * N) // NW


def _sc_edge_body(tab_hbm, ed_hbm, out_hbm, tab_v, idx_v, slab_v):
    wid = lax.axis_index("s") * 2 + lax.axis_index("c")
    pltpu.sync_copy(tab_hbm, tab_v)
    lane = lax.iota(jnp.int32, 16)
    lane8 = lane * 8

    def row_body(r, _):
        row = wid * ROWS_PER_W + r
        b = row // N
        i = row % N
        pltpu.sync_copy(ed_hbm.at[row], idx_v)

        def g_body(g):
            d32 = [
                plsc.load_gather(idx_v, [lane8 + (g * 128 + k)]) * H
                for k in range(K)
            ]

            def h_body(h):
                a = [plsc.load_gather(tab_v, [d32[k] + h])
                     for k in range(K)]
                s = ((a[0] + a[1]) + (a[2] + a[3])) + (
                    (a[4] + a[5]) + (a[6] + a[7]))
                plsc.store_scatter(
                    slab_v,
                    [jnp.full((16,), h, jnp.int32), lane + g * 16],
                    s * 0.125,
                )

            plsc.parallel_loop(0, H, 1, unroll=4)(h_body)

        plsc.parallel_loop(0, N // 16, 1)(g_body)
        pltpu.sync_copy(slab_v, out_hbm.at[b, :, i, :])
        return 0

    lax.fori_loop(0, ROWS_PER_W, row_body, 0)


@jax.jit
def _sc_edge(tab_flat, ed_rows):
    mesh = plsc.VectorSubcoreMesh(core_axis_name="c", subcore_axis_name="s")
    return pl.kernel(
        _sc_edge_body,
        out_type=jax.ShapeDtypeStruct((B, H, N, N), jnp.float32),
        mesh=mesh,
        compiler_params=pltpu.CompilerParams(needs_layout_passes=False),
        scratch_types=[
            pltpu.VMEM((V * H,), jnp.float32),
            pltpu.VMEM((N * K,), jnp.int32),
            pltpu.VMEM((H, N), jnp.float32),
        ],
    )(tab_flat, ed_rows)


def _tc_assemble_body(ab_ref, e_ref, o_ref):
    ab = ab_ref[0]
    e = e_ref[0]
    base = 2.0 * jnp.broadcast_to(ab[None, :, :], (H, N + 1, N + 1))
    row0 = jnp.zeros((H, 1, N), jnp.float32)
    col0 = jnp.zeros((H, N + 1, 1), jnp.float32)
    padded = jnp.concatenate(
        [col0, jnp.concatenate([row0, e], axis=1)], axis=2)
    o_ref[0] = base + padded


@jax.jit
def _tc_assemble(attn_bias, edge):
    return pl.pallas_call(
        _tc_assemble_body,
        out_shape=jax.ShapeDtypeStruct((B, H, N + 1, N + 1), jnp.float32),
        grid=(B,),
        in_specs=[
            pl.BlockSpec((1, N + 1, N + 1), lambda b: (b, 0, 0)),
            pl.BlockSpec((1, H, N, N), lambda b: (b, 0, 0, 0)),
        ],
        out_specs=pl.BlockSpec((1, H, N + 1, N + 1), lambda b: (b, 0, 0, 0)),
    )(attn_bias, edge)


def kernel(attn_bias, edge_data, edge_encoder_weight):
    ed = edge_data.astype(jnp.int32).reshape(B * N, N * K)
    tab = edge_encoder_weight.astype(jnp.float32).reshape(V * H)
    edge = _sc_edge(tab, ed)
    return _tc_assemble(attn_bias, edge)


# speedup vs baseline: 26.3359x; 4.0620x over previous
import functools

import jax
import jax.numpy as jnp
from jax import lax
from jax.experimental import pallas as pl
from jax.experimental.pallas import tpu as pltpu
from jax.experimental.pallas import tpu_sc as plsc

B, N, K, H, V = 8, 128, 8, 32, 512
NW = 32
ROWS_PER_W = (B * N) // NW


def _sc_edge_body(tab_hbm, ed_hbm, out_hbm, tab_v, idx_v, slab_v):
    wid = lax.axis_index("s") * 2 + lax.axis_index("c")
    pltpu.sync_copy(tab_hbm, tab_v)
    lane = lax.iota(jnp.int32, 16)

    def row_body(r, _):
        row = wid * ROWS_PER_W + r
        b = row // N
        i = row % N
        pltpu.sync_copy(ed_hbm.at[row], idx_v)

        def g_body(g):
            d = [idx_v[pl.ds(k * N + g * 16, 16)] for k in range(K)]

            def h_body(h):
                a = [plsc.load_gather(tab_v, [d[k] + h * V])
                     for k in range(K)]
                s = ((a[0] + a[1]) + (a[2] + a[3])) + (
                    (a[4] + a[5]) + (a[6] + a[7]))
                plsc.store_scatter(
                    slab_v,
                    [jnp.full((16,), h, jnp.int32), lane + g * 16],
                    s * 0.125,
                )

            plsc.parallel_loop(0, H, 1, unroll=4)(h_body)

        plsc.parallel_loop(0, N // 16, 1)(g_body)
        pltpu.sync_copy(slab_v, out_hbm.at[b, :, i, :])
        return 0

    lax.fori_loop(0, ROWS_PER_W, row_body, 0)


@jax.jit
def _sc_edge(tab_flat, ed_rows):
    mesh = plsc.VectorSubcoreMesh(core_axis_name="c", subcore_axis_name="s")
    return pl.kernel(
        _sc_edge_body,
        out_type=jax.ShapeDtypeStruct((B, H, N, N), jnp.float32),
        mesh=mesh,
        compiler_params=pltpu.CompilerParams(needs_layout_passes=False),
        scratch_types=[
            pltpu.VMEM((V * H,), jnp.float32),
            pltpu.VMEM((N * K,), jnp.int32),
            pltpu.VMEM((H, N), jnp.float32),
        ],
    )(tab_flat, ed_rows)


def _tc_assemble_body(ab_ref, e_ref, o_ref):
    ab = ab_ref[0]
    e = e_ref[0]
    base = 2.0 * jnp.broadcast_to(ab[None, :, :], (H, N + 1, N + 1))
    row0 = jnp.zeros((H, 1, N), jnp.float32)
    col0 = jnp.zeros((H, N + 1, 1), jnp.float32)
    padded = jnp.concatenate(
        [col0, jnp.concatenate([row0, e], axis=1)], axis=2)
    o_ref[0] = base + padded


@jax.jit
def _tc_assemble(attn_bias, edge):
    return pl.pallas_call(
        _tc_assemble_body,
        out_shape=jax.ShapeDtypeStruct((B, H, N + 1, N + 1), jnp.float32),
        grid=(B,),
        in_specs=[
            pl.BlockSpec((1, N + 1, N + 1), lambda b: (b, 0, 0)),
            pl.BlockSpec((1, H, N, N), lambda b: (b, 0, 0, 0)),
        ],
        out_specs=pl.BlockSpec((1, H, N + 1, N + 1), lambda b: (b, 0, 0, 0)),
    )(attn_bias, edge)


def kernel(attn_bias, edge_data, edge_encoder_weight):
    ed = jnp.transpose(edge_data.astype(jnp.int32), (0, 1, 3, 2)).reshape(
        B * N, K * N)
    tab = edge_encoder_weight.astype(jnp.float32).T.reshape(H * V)
    edge = _sc_edge(tab, ed)
    return _tc_assemble(attn_bias, edge)
